# Initial kernel scaffold; baseline (speedup 1.0000x reference)
import jax
import jax.numpy as jnp
from jax.experimental import pallas as pl


def kernel(point, feat, row_splits, W, gamma, beta):
    raise NotImplementedError("write your pallas kernel here")


# speedup vs baseline: 7.4108x; 7.4108x over previous
import functools

import jax
import jax.numpy as jnp
from jax import lax
from jax.experimental import pallas as pl
from jax.experimental.pallas import tpu as pltpu
from jax.experimental.pallas import tpu_sc as plsc

_B = 4
_NPC = 8192
_STRIDE = 4
_MPC = _NPC // _STRIDE
_NS = 16
_CIN = 64
_COUT = 128
_SUB = 8
_LAN = _NPC // _SUB
_QB = 128
_MB = 64

_BIGI = 2**30


def _fps_body(px_ref, py_ref, pz_ref, idx_ref, x_ref, y_ref, z_ref, dist_ref):
    fidx = (lax.broadcasted_iota(jnp.int32, (_SUB, _LAN), 0) * _LAN
            + lax.broadcasted_iota(jnp.int32, (_SUB, _LAN), 1))

    carry0 = []
    for b in range(_B):
        dist_ref[b] = jnp.full((_SUB, _LAN), jnp.inf, jnp.float32)
        idx_ref[b, 0:1, :] = jnp.zeros((1, 1), jnp.int32)
        x0 = jnp.sum(px_ref[b, 0:1, 0:1])
        y0 = jnp.sum(py_ref[b, 0:1, 0:1])
        z0 = jnp.sum(pz_ref[b, 0:1, 0:1])
        x_ref[b, 0:1, :] = jnp.broadcast_to(x0, (1, 1))
        y_ref[b, 0:1, :] = jnp.broadcast_to(y0, (1, 1))
        z_ref[b, 0:1, :] = jnp.broadcast_to(z0, (1, 1))
        carry0 += [x0, y0, z0]

    def step(i, carry):
        nxt = []
        for b in range(_B):
            lx, ly, lz = carry[3 * b:3 * b + 3]
            px, py, pz = px_ref[b], py_ref[b], pz_ref[b]
            dx = px - lx
            dy = py - ly
            dz = pz - lz
            d = (dx * dx + dy * dy) + dz * dz
            dn = jnp.minimum(dist_ref[b], d)
            dist_ref[b] = dn
            m = jnp.max(dn)
            j = jnp.min(jnp.where(dn == m, fidx, _BIGI))
            sel = fidx == j
            sx = jnp.max(jnp.where(sel, px, -jnp.inf))
            sy = jnp.max(jnp.where(sel, py, -jnp.inf))
            sz = jnp.max(jnp.where(sel, pz, -jnp.inf))
            idx_ref[b, pl.ds(i, 1), :] = jnp.broadcast_to(j, (1, 1))
            x_ref[b, pl.ds(i, 1), :] = jnp.broadcast_to(sx, (1, 1))
            y_ref[b, pl.ds(i, 1), :] = jnp.broadcast_to(sy, (1, 1))
            z_ref[b, pl.ds(i, 1), :] = jnp.broadcast_to(sz, (1, 1))
            nxt += [sx, sy, sz]
        return tuple(nxt)

    lax.fori_loop(1, _MPC, step, tuple(carry0))


def _fps_call(px, py, pz):
    out_shape = [
        jax.ShapeDtypeStruct((_B, _MPC, 1), jnp.int32),
        jax.ShapeDtypeStruct((_B, _MPC, 1), jnp.float32),
        jax.ShapeDtypeStruct((_B, _MPC, 1), jnp.float32),
        jax.ShapeDtypeStruct((_B, _MPC, 1), jnp.float32),
    ]
    return pl.pallas_call(
        _fps_body,
        out_shape=out_shape,
        scratch_shapes=[pltpu.VMEM((_B, _SUB, _LAN), jnp.float32)],
    )(px, py, pz)


def _knn_body(qx_ref, qy_ref, qz_ref, px_ref, py_ref, pz_ref, out_ref):
    b = pl.program_id(0)
    qx = qx_ref[0].reshape(_QB, 1)
    qy = qy_ref[0].reshape(_QB, 1)
    qz = qz_ref[0].reshape(_QB, 1)
    px = px_ref[...].reshape(1, _NPC)
    py = py_ref[...].reshape(1, _NPC)
    pz = pz_ref[...].reshape(1, _NPC)
    dx = qx - px
    dy = qy - py
    dz = qz - pz
    d = (dx * dx + dy * dy) + dz * dz
    col = lax.broadcasted_iota(jnp.int32, (_QB, _NPC), 1)
    picks = []
    for _ in range(_NS):
        m = jnp.min(d, axis=1, keepdims=True)
        c = jnp.min(jnp.where(d == m, col, _BIGI), axis=1, keepdims=True)
        picks.append(c)
        d = jnp.where(col == c, jnp.inf, d)
    out_ref[0] = jnp.concatenate(picks, axis=1) + b * _NPC


def _knn_call(qx, qy, qz, px, py, pz):
    nqb = _MPC // _QB
    grid = (_B, nqb)
    qspec = pl.BlockSpec((1, _QB, 1), lambda b, n: (b, n, 0))
    pspec = pl.BlockSpec((1, 1, _NPC), lambda b, n: (b, 0, 0))
    return pl.pallas_call(
        _knn_body,
        grid=grid,
        in_specs=[qspec, qspec, qspec, pspec, pspec, pspec],
        out_specs=pl.BlockSpec((1, _QB, _NS), lambda b, n: (b, n, 0)),
        out_shape=jax.ShapeDtypeStruct((_B, _MPC, _NS), jnp.int32),
    )(qx, qy, qz, px, py, pz)


_NROWS = _B * _MPC * _NS
_CHUNK = 128
_TW = 128


def _sc_gather_call(table, idx_flat):
    info = plsc.get_sparse_core_info()
    nc, nsub = info.num_cores, info.num_subcores
    nw = nc * nsub
    rows_per_w = _NROWS // nw
    nchunk = rows_per_w // _CHUNK
    mesh = plsc.VectorSubcoreMesh(core_axis_name="c", subcore_axis_name="s")

    @functools.partial(
        pl.kernel,
        mesh=mesh,
        out_type=jax.ShapeDtypeStruct((_NROWS, _TW), jnp.float32),
        scratch_types=[
            pltpu.VMEM((_CHUNK,), jnp.int32),
            pltpu.VMEM((_CHUNK, _TW), jnp.float32),
            pltpu.SemaphoreType.DMA,
        ],
    )
    def k(tab_hbm, idx_hbm, out_hbm, idx_v, rows_v, sem):
        wid = lax.axis_index("s") * nc + lax.axis_index("c")
        base_w = wid * rows_per_w

        def chunk(t, _):
            base = pl.multiple_of(base_w + t * _CHUNK, _CHUNK)
            pltpu.sync_copy(idx_hbm.at[pl.ds(base, _CHUNK)], idx_v)
            pltpu.async_copy(tab_hbm.at[idx_v], rows_v, sem).wait()
            pltpu.sync_copy(rows_v, out_hbm.at[pl.ds(base, _CHUNK)])
            return 0

        lax.fori_loop(0, nchunk, chunk, 0)

    return k(table, idx_flat)


def _mlp_body(g_ref, q_ref, w_ref, sc_ref, be_ref, out_ref):
    h = jnp.dot(g_ref[...], w_ref[...], preferred_element_type=jnp.float32)
    q = q_ref[...]
    w3 = w_ref[...]
    hq = (q[:, 0:1] * w3[0:1, :] + q[:, 1:2] * w3[1:2, :]
          + q[:, 2:3] * w3[2:3, :])
    h = h.reshape(_MB, _NS, _COUT) - hq[:, None, :]
    h = h * sc_ref[...][None] + be_ref[...][None]
    h = jnp.maximum(h, 0.0)
    out_ref[...] = jnp.max(h, axis=1)


def _mlp_call(grows, q3, wpad, scale, beta2):
    nblk = (_B * _MPC) // _MB
    rows = _MB * _NS
    return pl.pallas_call(
        _mlp_body,
        grid=(nblk,),
        in_specs=[
            pl.BlockSpec((rows, _TW), lambda i: (i, 0)),
            pl.BlockSpec((_MB, 3), lambda i: (i, 0)),
            pl.BlockSpec((_TW, _COUT), lambda i: (0, 0)),
            pl.BlockSpec((1, _COUT), lambda i: (0, 0)),
            pl.BlockSpec((1, _COUT), lambda i: (0, 0)),
        ],
        out_specs=pl.BlockSpec((_MB, _COUT), lambda i: (i, 0)),
        out_shape=jax.ShapeDtypeStruct((_B * _MPC, _COUT), jnp.float32),
    )(grows, q3, wpad, scale, beta2)


def kernel(point, feat, row_splits, W, gamma, beta):
    px = point[:, 0].reshape(_B, _SUB, _LAN)
    py = point[:, 1].reshape(_B, _SUB, _LAN)
    pz = point[:, 2].reshape(_B, _SUB, _LAN)

    idx, sx, sy, sz = _fps_call(px, py, pz)

    pxr = point[:, 0].reshape(_B, 1, _NPC)
    pyr = point[:, 1].reshape(_B, 1, _NPC)
    pzr = point[:, 2].reshape(_B, 1, _NPC)
    idx_knn = _knn_call(sx, sy, sz, pxr, pyr, pzr)

    table = jnp.concatenate(
        [point, feat, jnp.zeros((_B * _NPC, _TW - 3 - _CIN), jnp.float32)],
        axis=1)
    grows = _sc_gather_call(table, idx_knn.reshape(-1))

    new_point = jnp.concatenate([sx, sy, sz], axis=2).reshape(_B * _MPC, 3)
    wpad = jnp.concatenate(
        [W, jnp.zeros((_TW - 3 - _CIN, _COUT), jnp.float32)], axis=0)
    scale = (gamma / jnp.sqrt(1.0 + 1e-5)).reshape(1, _COUT)
    beta2 = beta.reshape(1, _COUT)
    out_feat = _mlp_call(grows, new_point, wpad, scale, beta2)

    new_row_splits = (jnp.arange(_B + 1) * _MPC).astype(jnp.int32)
    return new_point, out_feat, new_row_splits


# speedup vs baseline: 14.9990x; 2.0239x over previous
import functools

import jax
import jax.numpy as jnp
from jax import lax
from jax.experimental import pallas as pl
from jax.experimental.pallas import tpu as pltpu
from jax.experimental.pallas import tpu_sc as plsc

_B = 4
_NPC = 8192
_STRIDE = 4
_MPC = _NPC // _STRIDE
_NS = 16
_CIN = 64
_COUT = 128
_SUB = 8
_LAN = _NPC // _SUB
_QB = 128
_MB = 64

_BIGI = 2**30


_FR = _NPC // 128


def _allmax_row(x):
    for s in (64, 32, 16, 8, 4, 2, 1):
        x = jnp.maximum(x, pltpu.roll(x, s, 1))
    return x


def _allmin_row(x):
    for s in (64, 32, 16, 8, 4, 2, 1):
        x = jnp.minimum(x, pltpu.roll(x, s, 1))
    return x


def _fps_body(px_ref, py_ref, pz_ref, idx_ref, x_ref, y_ref, z_ref,
              d0_ref, d1_ref, d2_ref, d3_ref):
    drefs = [d0_ref, d1_ref, d2_ref, d3_ref]
    fidxf = (lax.broadcasted_iota(jnp.int32, (_FR, 128), 0) * 128
             + lax.broadcasted_iota(jnp.int32, (_FR, 128), 1)
             ).astype(jnp.float32)

    carry0 = []
    for b in range(_B):
        drefs[b][...] = jnp.full((_FR, 128), jnp.inf, jnp.float32)
        idx_ref[b, 0:1, :] = jnp.zeros((1, 1), jnp.int32)
        x0 = jnp.broadcast_to(px_ref[b, 0:1, 0:1], (1, 128))
        y0 = jnp.broadcast_to(py_ref[b, 0:1, 0:1], (1, 128))
        z0 = jnp.broadcast_to(pz_ref[b, 0:1, 0:1], (1, 128))
        x_ref[b, 0:1, :] = x0[:, 0:1]
        y_ref[b, 0:1, :] = y0[:, 0:1]
        z_ref[b, 0:1, :] = z0[:, 0:1]
        carry0 += [x0, y0, z0]

    rowi = lax.broadcasted_iota(jnp.int32, (_FR, 128), 0).astype(jnp.float32)
    lanei = lax.broadcasted_iota(jnp.int32, (1, 128), 1).astype(jnp.float32)

    def step(i, carry):
        colmax = []
        keys = []
        crows = []
        for b in range(_B):
            lx, ly, lz = carry[3 * b:3 * b + 3]
            px, py, pz = px_ref[b], py_ref[b], pz_ref[b]
            dx = px - lx
            dy = py - ly
            dz = pz - lz
            d = (dx * dx + dy * dy) + dz * dz
            dn = jnp.minimum(drefs[b][...], d)
            drefs[b][...] = dn
            cm = jnp.max(dn, axis=0, keepdims=True)
            rfirst = jnp.min(jnp.where(dn == cm, rowi, jnp.inf),
                             axis=0, keepdims=True)
            rsel = rowi == rfirst
            colmax.append(cm)
            keys.append(rfirst * 128.0 + lanei)
            crows.append(jnp.max(jnp.where(rsel, px, -jnp.inf),
                                 axis=0, keepdims=True))
            crows.append(jnp.max(jnp.where(rsel, py, -jnp.inf),
                                 axis=0, keepdims=True))
            crows.append(jnp.max(jnp.where(rsel, pz, -jnp.inf),
                                 axis=0, keepdims=True))
        m4c = jnp.concatenate(colmax, axis=0)
        key4 = jnp.concatenate(keys, axis=0)
        m4 = jnp.max(m4c, axis=1, keepdims=True)
        m4b = jnp.broadcast_to(m4, (_B, 128))
        j4 = jnp.min(jnp.where(m4c == m4b, key4, jnp.inf),
                     axis=1, keepdims=True)
        j4b = jnp.broadcast_to(j4, (_B, 128))
        lsel = key4 == j4b
        c12 = jnp.concatenate(crows, axis=0)
        nxt = []
        for b in range(_B):
            sb = lsel[b:b + 1]
            cx = jnp.max(jnp.where(sb, c12[3 * b:3 * b + 1], -jnp.inf),
                         axis=1, keepdims=True)
            cy = jnp.max(jnp.where(sb, c12[3 * b + 1:3 * b + 2], -jnp.inf),
                         axis=1, keepdims=True)
            cz = jnp.max(jnp.where(sb, c12[3 * b + 2:3 * b + 3], -jnp.inf),
                         axis=1, keepdims=True)
            cxb = jnp.broadcast_to(cx, (1, 128))
            cyb = jnp.broadcast_to(cy, (1, 128))
            czb = jnp.broadcast_to(cz, (1, 128))
            idx_ref[b, pl.ds(i, 1), :] = j4[b:b + 1].astype(jnp.int32)
            x_ref[b, pl.ds(i, 1), :] = cx
            y_ref[b, pl.ds(i, 1), :] = cy
            z_ref[b, pl.ds(i, 1), :] = cz
            nxt += [cxb, cyb, czb]
        return tuple(nxt)

    lax.fori_loop(1, _MPC, step, tuple(carry0))


def _fps_call(px, py, pz):
    out_shape = [
        jax.ShapeDtypeStruct((_B, _MPC, 1), jnp.int32),
        jax.ShapeDtypeStruct((_B, _MPC, 1), jnp.float32),
        jax.ShapeDtypeStruct((_B, _MPC, 1), jnp.float32),
        jax.ShapeDtypeStruct((_B, _MPC, 1), jnp.float32),
    ]
    return pl.pallas_call(
        _fps_body,
        out_shape=out_shape,
        scratch_shapes=[pltpu.VMEM((_FR, 128), jnp.float32)
                        for _ in range(_B)],
    )(px, py, pz)


def _knn_body(qx_ref, qy_ref, qz_ref, px_ref, py_ref, pz_ref, out_ref):
    b = pl.program_id(0)
    qx = qx_ref[0].reshape(_QB, 1)
    qy = qy_ref[0].reshape(_QB, 1)
    qz = qz_ref[0].reshape(_QB, 1)
    px = px_ref[...].reshape(1, _NPC)
    py = py_ref[...].reshape(1, _NPC)
    pz = pz_ref[...].reshape(1, _NPC)
    dx = qx - px
    dy = qy - py
    dz = qz - pz
    d = (dx * dx + dy * dy) + dz * dz
    col = lax.broadcasted_iota(jnp.int32, (_QB, _NPC), 1)
    picks = []
    for _ in range(_NS):
        m = jnp.min(d, axis=1, keepdims=True)
        c = jnp.min(jnp.where(d == m, col, _BIGI), axis=1, keepdims=True)
        picks.append(c)
        d = jnp.where(col == c, jnp.inf, d)
    out_ref[0] = jnp.concatenate(picks, axis=1) + b * _NPC


def _knn_call(qx, qy, qz, px, py, pz):
    nqb = _MPC // _QB
    grid = (_B, nqb)
    qspec = pl.BlockSpec((1, _QB, 1), lambda b, n: (b, n, 0))
    pspec = pl.BlockSpec((1, 1, _NPC), lambda b, n: (b, 0, 0))
    return pl.pallas_call(
        _knn_body,
        grid=grid,
        in_specs=[qspec, qspec, qspec, pspec, pspec, pspec],
        out_specs=pl.BlockSpec((1, _QB, _NS), lambda b, n: (b, n, 0)),
        out_shape=jax.ShapeDtypeStruct((_B, _MPC, _NS), jnp.int32),
    )(qx, qy, qz, px, py, pz)


_NROWS = _B * _MPC * _NS
_CHUNK = 128
_TW = 128


def _sc_gather_call(table, idx_flat):
    info = plsc.get_sparse_core_info()
    nc, nsub = info.num_cores, info.num_subcores
    nw = nc * nsub
    rows_per_w = _NROWS // nw
    nchunk = rows_per_w // _CHUNK
    mesh = plsc.VectorSubcoreMesh(core_axis_name="c", subcore_axis_name="s")

    @functools.partial(
        pl.kernel,
        mesh=mesh,
        out_type=jax.ShapeDtypeStruct((_NROWS, _TW), jnp.float32),
        scratch_types=[
            pltpu.VMEM((_CHUNK,), jnp.int32),
            pltpu.VMEM((_CHUNK, _TW), jnp.float32),
            pltpu.SemaphoreType.DMA,
        ],
    )
    def k(tab_hbm, idx_hbm, out_hbm, idx_v, rows_v, sem):
        wid = lax.axis_index("s") * nc + lax.axis_index("c")
        base_w = wid * rows_per_w

        def chunk(t, _):
            base = pl.multiple_of(base_w + t * _CHUNK, _CHUNK)
            pltpu.sync_copy(idx_hbm.at[pl.ds(base, _CHUNK)], idx_v)
            pltpu.async_copy(tab_hbm.at[idx_v], rows_v, sem).wait()
            pltpu.sync_copy(rows_v, out_hbm.at[pl.ds(base, _CHUNK)])
            return 0

        lax.fori_loop(0, nchunk, chunk, 0)

    return k(table, idx_flat)


def _mlp_body(g_ref, q_ref, w_ref, sc_ref, be_ref, out_ref):
    h = jnp.dot(g_ref[...], w_ref[...], preferred_element_type=jnp.float32)
    q = q_ref[...]
    w3 = w_ref[...]
    hq = (q[:, 0:1] * w3[0:1, :] + q[:, 1:2] * w3[1:2, :]
          + q[:, 2:3] * w3[2:3, :])
    h = h.reshape(_MB, _NS, _COUT) - hq[:, None, :]
    h = h * sc_ref[...][None] + be_ref[...][None]
    h = jnp.maximum(h, 0.0)
    out_ref[...] = jnp.max(h, axis=1)


def _mlp_call(grows, q3, wpad, scale, beta2):
    nblk = (_B * _MPC) // _MB
    rows = _MB * _NS
    return pl.pallas_call(
        _mlp_body,
        grid=(nblk,),
        in_specs=[
            pl.BlockSpec((rows, _TW), lambda i: (i, 0)),
            pl.BlockSpec((_MB, 3), lambda i: (i, 0)),
            pl.BlockSpec((_TW, _COUT), lambda i: (0, 0)),
            pl.BlockSpec((1, _COUT), lambda i: (0, 0)),
            pl.BlockSpec((1, _COUT), lambda i: (0, 0)),
        ],
        out_specs=pl.BlockSpec((_MB, _COUT), lambda i: (i, 0)),
        out_shape=jax.ShapeDtypeStruct((_B * _MPC, _COUT), jnp.float32),
    )(grows, q3, wpad, scale, beta2)


def kernel(point, feat, row_splits, W, gamma, beta):
    px = point[:, 0].reshape(_B, _FR, 128)
    py = point[:, 1].reshape(_B, _FR, 128)
    pz = point[:, 2].reshape(_B, _FR, 128)

    idx, sx, sy, sz = _fps_call(px, py, pz)

    pxr = point[:, 0].reshape(_B, 1, _NPC)
    pyr = point[:, 1].reshape(_B, 1, _NPC)
    pzr = point[:, 2].reshape(_B, 1, _NPC)
    idx_knn = _knn_call(sx, sy, sz, pxr, pyr, pzr)

    table = jnp.concatenate(
        [point, feat, jnp.zeros((_B * _NPC, _TW - 3 - _CIN), jnp.float32)],
        axis=1)
    grows = _sc_gather_call(table, idx_knn.reshape(-1))

    new_point = jnp.concatenate([sx, sy, sz], axis=2).reshape(_B * _MPC, 3)
    wpad = jnp.concatenate(
        [W, jnp.zeros((_TW - 3 - _CIN, _COUT), jnp.float32)], axis=0)
    scale = (gamma / jnp.sqrt(1.0 + 1e-5)).reshape(1, _COUT)
    beta2 = beta.reshape(1, _COUT)
    out_feat = _mlp_call(grows, new_point, wpad, scale, beta2)

    new_row_splits = (jnp.arange(_B + 1) * _MPC).astype(jnp.int32)
    return new_point, out_feat, new_row_splits
